# baseline (speedup 1.0000x reference)
import jax
import jax.numpy as jnp
from jax.experimental import pallas as pl
from jax.experimental.pallas import tpu as pltpu

_D = 4
_E = 4


def _compute(ibuf, obuf, islot, oslot):
    d = jax.lax.broadcasted_iota(jnp.int32, (32, 32, 128), 2)
    src = (d // 64) * 64 + 2 * (d % 32) + (d % 64) // 32
    for rh in range(2):
        vh = ibuf[islot, pl.ds(rh, 32, 2), :, :]
        g1 = jnp.take_along_axis(vh, src, axis=-1)
        for p in range(4):
            t = g1[:, :, 32 * p : 32 * p + 32]
            obuf[oslot, 4 * rh + p] = t.reshape(32, 1024)


def _body(x_hbm, o_hbm, ibuf, obuf, isem, osem):
    i = pl.program_id(0)
    n = pl.num_programs(0)

    def in_copy(j):
        return pltpu.make_async_copy(
            x_hbm.at[j], ibuf.at[jax.lax.rem(j, _D)], isem.at[jax.lax.rem(j, _D)]
        )

    def out_copy(j):
        return pltpu.make_async_copy(
            obuf.at[jax.lax.rem(j, _E)], o_hbm.at[j], osem.at[jax.lax.rem(j, _E)]
        )

    @pl.when(i == 0)
    def _():
        for j in range(_D):
            in_copy(jnp.int32(j)).start()

    @pl.when((i > 0) & (i + _D - 1 < n))
    def _():
        in_copy(i + _D - 1).start()

    in_copy(i).wait()

    @pl.when(i >= _E)
    def _():
        out_copy(i - _E).wait()

    _compute(ibuf, obuf, jax.lax.rem(i, _D), jax.lax.rem(i, _E))

    out_copy(i).start()

    @pl.when(i == n - 1)
    def _():
        for k in range(_E):
            out_copy(n - _E + k).wait()


def kernel(x):
    B, C, H, W, Z = x.shape
    r = 2
    N = B * C
    x2 = x.reshape(N, H, W // r, r * Z)
    out = pl.pallas_call(
        _body,
        grid=(N,),
        in_specs=[pl.BlockSpec(memory_space=pltpu.MemorySpace.HBM)],
        out_specs=pl.BlockSpec(memory_space=pltpu.MemorySpace.HBM),
        out_shape=jax.ShapeDtypeStruct(
            (N, r**3, H // r, (W // r) * (Z // r)), x.dtype
        ),
        scratch_shapes=[
            pltpu.VMEM((_D, H, W // r, r * Z), x.dtype),
            pltpu.VMEM((_E, r**3, H // r, (W // r) * (Z // r)), x.dtype),
            pltpu.SemaphoreType.DMA((_D,)),
            pltpu.SemaphoreType.DMA((_E,)),
        ],
    )(x2)
    return out.reshape(B, C * r**3, H // r, W // r, Z // r)


# speedup vs baseline: 1.0078x; 1.0078x over previous
import jax
import jax.numpy as jnp
from jax.experimental import pallas as pl


def _permute_cube(v_ref, out_ref, half):
    d = jax.lax.broadcasted_iota(jnp.int32, (32, 32, 128), 2)
    src = (d // 64) * 64 + 2 * (d % 32) + (d % 64) // 32
    for rh in range(2):
        vh = v_ref[0, 0, pl.ds(rh, 32, 2), :, :]
        g1 = jnp.take_along_axis(vh, src, axis=-1)
        for p in range(4):
            t = g1[:, :, 32 * p : 32 * p + 32]
            out_ref[half, 0, 4 * rh + p] = t.reshape(32, 1024)


def _body(xa_ref, xb_ref, o_ref):
    _permute_cube(xa_ref, o_ref, 0)
    _permute_cube(xb_ref, o_ref, 1)


def kernel(x):
    B, C, H, W, Z = x.shape
    r = 2
    x2 = x.reshape(B, C, H, W // r, r * Z)
    out = pl.pallas_call(
        _body,
        grid=(C,),
        in_specs=[
            pl.BlockSpec((1, 1, H, W // r, r * Z), lambda i: (0, i, 0, 0, 0)),
            pl.BlockSpec((1, 1, H, W // r, r * Z), lambda i: (1, i, 0, 0, 0)),
        ],
        out_specs=pl.BlockSpec(
            (B, 1, r**3, H // r, (W // r) * (Z // r)),
            lambda i: (0, i, 0, 0, 0),
        ),
        out_shape=jax.ShapeDtypeStruct(
            (B, C, r**3, H // r, (W // r) * (Z // r)), x.dtype
        ),
    )(x2, x2)
    return out.reshape(B, C * r**3, H // r, W // r, Z // r)


# speedup vs baseline: 1.8298x; 1.8157x over previous
import jax
import jax.numpy as jnp
from jax import lax
from jax.experimental import pallas as pl
from jax.experimental.pallas import tpu as pltpu
from jax.experimental.pallas import tpu_sc as plsc

_UNITS = 64


def _unit(x_hbm, o_hbm, ib, ob, isem, osem, wid, t, do_wait_out):
    cube = wid * 4 + lax.div(t, 32)
    ho = lax.rem(t, 32)

    pltpu.make_async_copy(x_hbm.at[cube, pl.ds(2 * ho, 2)], ib, isem).wait()

    @pl.when(do_wait_out)
    def _():
        pltpu.make_async_copy(
            ob, o_hbm.at[cube, :, pl.ds(ho, 1)], osem
        ).wait()

    two_iota = 2 * lax.iota(jnp.int32, 16)

    @pl.loop(0, 32)
    def _(wo):
        for p in range(8):
            rh, rw, rz = p // 4, (p // 2) % 2, p % 2
            d0 = jnp.full((16,), rh, jnp.int32)
            w_idx = jnp.full((16,), 2 * wo + rw, jnp.int32)
            for k in range(2):
                z_idx = (rz + 32 * k) + two_iota
                vec = plsc.load_gather(ib, [d0, w_idx, z_idx])
                ob[p, 0, wo, pl.ds(16 * k, 16)] = vec

    pltpu.make_async_copy(ob, o_hbm.at[cube, :, pl.ds(ho, 1)], osem).start()

    t2 = t + 2

    @pl.when(t2 < 2 * _UNITS)
    def _():
        cube2 = wid * 4 + lax.div(t2, 32)
        ho2 = lax.rem(t2, 32)
        pltpu.make_async_copy(
            x_hbm.at[cube2, pl.ds(2 * ho2, 2)], ib, isem
        ).start()


def _sc_body(x_hbm, o_hbm, ib0, ib1, ob0, ob1, is0, is1, os0, os1):
    wid = lax.axis_index("s") * 2 + lax.axis_index("c")

    pltpu.make_async_copy(x_hbm.at[wid * 4, pl.ds(0, 2)], ib0, is0).start()
    pltpu.make_async_copy(x_hbm.at[wid * 4, pl.ds(2, 2)], ib1, is1).start()

    @pl.loop(0, _UNITS)
    def _(j):
        t = 2 * j
        _unit(x_hbm, o_hbm, ib0, ob0, is0, os0, wid, t, j >= 1)
        _unit(x_hbm, o_hbm, ib1, ob1, is1, os1, wid, t + 1, j >= 1)

    cube_l = wid * 4 + 3
    pltpu.make_async_copy(ob0, o_hbm.at[cube_l, :, pl.ds(30, 1)], os0).wait()
    pltpu.make_async_copy(ob1, o_hbm.at[cube_l, :, pl.ds(31, 1)], os1).wait()


def kernel(x):
    B, C, H, W, Z = x.shape
    r = 2
    N = B * C
    x2 = x.reshape(N, H, W, Z)
    mesh = plsc.VectorSubcoreMesh(core_axis_name="c", subcore_axis_name="s")
    f = pl.kernel(
        _sc_body,
        out_type=jax.ShapeDtypeStruct(
            (N, r**3, H // r, W // r, Z // r), x.dtype
        ),
        mesh=mesh,
        compiler_params=pltpu.CompilerParams(needs_layout_passes=False),
        scratch_types=[
            pltpu.VMEM((2, W, Z), x.dtype),
            pltpu.VMEM((2, W, Z), x.dtype),
            pltpu.VMEM((r**3, 1, W // r, Z // r), x.dtype),
            pltpu.VMEM((r**3, 1, W // r, Z // r), x.dtype),
            pltpu.SemaphoreType.DMA,
            pltpu.SemaphoreType.DMA,
            pltpu.SemaphoreType.DMA,
            pltpu.SemaphoreType.DMA,
        ],
    )
    out = f(x2)
    return out.reshape(B, C * r**3, H // r, W // r, Z // r)
